# baseline (speedup 1.0000x reference)
import functools

import jax
import jax.numpy as jnp
from jax import lax
from jax.experimental import pallas as pl
from jax.experimental.pallas import tpu as pltpu
from jax.experimental.pallas import tpu_sc as plsc

N = 10000
E = 320000
DF = 128
D = 32
OUT = 300
G = 256
BN_EPS = 1e-5

NC, NS = 2, 16
NW = NC * NS
P = 10240
ZROW = N
TRASH = N + 8
CHUNK = 128
NCHUNK = 80
EPAD = NW * NCHUNK * CHUNK
RPS = P // NS
FB = 2048

PG = 512
GTRASH = G + 8
CHUNK_P = 64
NCH_P = P // (NW * CHUNK_P)


def _sc_fold(tab, edges):
    mesh = plsc.VectorSubcoreMesh(core_axis_name="c", subcore_axis_name="s")

    @functools.partial(
        pl.kernel,
        mesh=mesh,
        out_type=jax.ShapeDtypeStruct((NC, P, D), jnp.float32),
        scratch_types=[
            pltpu.VMEM((NCHUNK, CHUNK), jnp.int32),
            pltpu.VMEM((NCHUNK, CHUNK), jnp.int32),
            pltpu.VMEM((CHUNK, D), jnp.float32),
            pltpu.VMEM((CHUNK, D), jnp.float32),
            pltpu.VMEM((FB, D), jnp.float32),
            pltpu.VMEM((FB // CHUNK, CHUNK), jnp.int32),
            pltpu.VMEM_SHARED((P, D), jnp.float32),
            pltpu.SemaphoreType.DMA,
            pltpu.SemaphoreType.DMA,
        ],
        compiler_params=pltpu.CompilerParams(use_tc_tiling_on_sc=False,
                                            needs_layout_passes=False),
    )
    def fold_kernel(tab_hbm, e_hbm, out_hbm, src_v, dst_v, rows_a, rows_b,
                    loc, fidx, agg_sh, sem_a, sem_b):
        c = lax.axis_index("c")
        s = lax.axis_index("s")
        wid = c * NS + s
        base = s * RPS
        z16 = jnp.zeros((16,), jnp.float32)

        def _zero_row(i, carry):
            loc[i, pl.ds(0, 16)] = z16
            loc[i, pl.ds(16, 16)] = z16
            return carry
        lax.fori_loop(0, FB, _zero_row, 0)

        pltpu.sync_copy(loc.at[pl.ds(0, RPS)], agg_sh.at[pl.ds(base, RPS)])
        pltpu.sync_copy(e_hbm.at[0, wid], src_v)
        pltpu.sync_copy(e_hbm.at[1, wid], dst_v)
        plsc.subcore_barrier()

        row_base = dst_v[0, pl.ds(0, 16)][0]

        iota16 = lax.iota(jnp.int32, 16)

        def _fill_ids(r, carry):
            for g in range(CHUNK // 16):
                fidx[r, pl.ds(g * 16, 16)] = jnp.minimum(
                    row_base + r * CHUNK + g * 16 + iota16, P - 1)
            return carry
        lax.fori_loop(0, FB // CHUNK, _fill_ids, 0)

        def gather(j, buf, sem):
            return pltpu.make_async_copy(tab_hbm.at[src_v.at[j]], buf, sem)

        def fold_chunk(j, rows, carry):
            def group_body(g, carry):
                d16 = dst_v[j, pl.ds(g * 16, 16)]

                def one_edge(e16, carry):
                    cur, acc0, acc1 = carry
                    e = g * 16 + e16
                    d = d16[e16]
                    r0 = rows[e, pl.ds(0, 16)]
                    r1 = rows[e, pl.ds(16, 16)]
                    same = d == cur
                    flush = jnp.logical_and(jnp.logical_not(same), cur >= 0)

                    @pl.when(flush)
                    def _():
                        rb16 = jnp.full((16,), cur - row_base, jnp.int32)
                        plsc.store_scatter(loc, [rb16, iota16], acc0)
                        plsc.store_scatter(loc, [rb16, iota16 + 16], acc1)

                    acc0 = jnp.where(same, acc0 + r0, r0)
                    acc1 = jnp.where(same, acc1 + r1, r1)
                    return d, acc0, acc1

                for e16 in range(16):
                    carry = one_edge(e16, carry)
                return carry
            return lax.fori_loop(0, CHUNK // 16, group_body, carry)

        gather(0, rows_a, sem_a).start()
        carry = (jnp.int32(-1), z16, z16)

        def chunk_pair(j2, carry):
            j = j2 * 2
            gather(j + 1, rows_b, sem_b).start()
            gather(j, rows_a, sem_a).wait()
            carry = fold_chunk(j, rows_a, carry)

            @pl.when(j2 < NCHUNK // 2 - 1)
            def _():
                gather(j + 2, rows_a, sem_a).start()

            gather(j + 1, rows_b, sem_b).wait()
            carry = fold_chunk(j + 1, rows_b, carry)
            return carry

        cur, acc0, acc1 = lax.fori_loop(0, NCHUNK // 2, chunk_pair, carry)

        @pl.when(cur >= 0)
        def _():
            rb16 = jnp.full((16,), cur - row_base, jnp.int32)
            plsc.store_scatter(loc, [rb16, iota16], acc0)
            plsc.store_scatter(loc, [rb16, iota16 + 16], acc1)

        span = cur - row_base + 1
        n_ops = (span + CHUNK - 1) // CHUNK

        def scat(k, carry):
            pltpu.sync_copy(loc.at[pl.ds(k * CHUNK, CHUNK)],
                            agg_sh.at[fidx.at[k]], add=True)
            return carry
        lax.fori_loop(0, n_ops, scat, 0)

        plsc.subcore_barrier()
        pltpu.sync_copy(agg_sh.at[pl.ds(base, RPS)],
                        out_hbm.at[c, pl.ds(base, RPS)])

    return fold_kernel(tab, edges)


def _sc_pool(h_tab, pool_dst):
    mesh = plsc.VectorSubcoreMesh(core_axis_name="c", subcore_axis_name="s")
    dpr = PG // NS

    @functools.partial(
        pl.kernel,
        mesh=mesh,
        out_type=jax.ShapeDtypeStruct((NC, PG, D), jnp.float32),
        scratch_types=[
            pltpu.VMEM((NCH_P, CHUNK_P), jnp.int32),
            pltpu.VMEM((CHUNK_P, D), jnp.float32),
            pltpu.VMEM((dpr, D), jnp.float32),
            pltpu.VMEM_SHARED((PG, D), jnp.float32),
        ],
        compiler_params=pltpu.CompilerParams(use_tc_tiling_on_sc=False),
    )
    def pool_kernel(h_hbm, pd_hbm, out_hbm, dst_v, rows_v, zbuf, agg_sh):
        c = lax.axis_index("c")
        s = lax.axis_index("s")
        wid = c * NS + s

        def _zero_row(i, carry):
            zbuf[i, pl.ds(0, 16)] = jnp.zeros((16,), jnp.float32)
            zbuf[i, pl.ds(16, 16)] = jnp.zeros((16,), jnp.float32)
            return carry
        lax.fori_loop(0, dpr, _zero_row, 0)

        pltpu.sync_copy(zbuf, agg_sh.at[pl.ds(s * dpr, dpr)])
        pltpu.sync_copy(pd_hbm.at[wid], dst_v)
        plsc.subcore_barrier()

        base = wid * (NCH_P * CHUNK_P)
        for j in range(NCH_P):
            pltpu.sync_copy(h_hbm.at[pl.ds(base + j * CHUNK_P, CHUNK_P)],
                            rows_v)
            pltpu.sync_copy(rows_v, agg_sh.at[dst_v.at[j]], add=True)

        plsc.subcore_barrier()
        pltpu.sync_copy(agg_sh.at[pl.ds(s * dpr, dpr)],
                        out_hbm.at[c, pl.ds(s * dpr, dpr)])

    return pool_kernel(h_tab, pool_dst)


def _tc_mlp1(x_pad, parts4, w1, b1, w2, b2):
    parts = jnp.concatenate(parts4, axis=2)

    def body(x_ref, parts_ref, w1_ref, b1_ref, w2_ref, b2_ref, o_ref):
        u = x_ref[...] + parts_ref[0] + parts_ref[1]
        a = jnp.maximum(jnp.dot(u, w1_ref[...],
                                preferred_element_type=jnp.float32)
                        + b1_ref[...], 0.0)
        o_ref[...] = jnp.maximum(
            jnp.dot(a, w2_ref[...], preferred_element_type=jnp.float32)
            + b2_ref[...], 0.0)
    return pl.pallas_call(
        body,
        out_shape=jax.ShapeDtypeStruct((P, D), jnp.float32),
    )(x_pad, parts, w1, b1.reshape(1, D), w2, b2.reshape(1, D))


def _tc_mlp(h, parts, w1, b1, w2, b2):
    def body(h_ref, parts_ref, w1_ref, b1_ref, w2_ref, b2_ref, o_ref):
        u = h_ref[...] + parts_ref[0] + parts_ref[1]
        a = jnp.maximum(jnp.dot(u, w1_ref[...],
                                preferred_element_type=jnp.float32)
                        + b1_ref[...], 0.0)
        o_ref[...] = jnp.maximum(
            jnp.dot(a, w2_ref[...], preferred_element_type=jnp.float32)
            + b2_ref[...], 0.0)
    return pl.pallas_call(
        body,
        out_shape=jax.ShapeDtypeStruct((P, D), jnp.float32),
    )(h, parts, w1, b1.reshape(1, D), w2, b2.reshape(1, D))


def _tc_norm(z, mean, var, gamma, beta):
    def body(z_ref, m_ref, v_ref, g_ref, be_ref, o_ref):
        h = (g_ref[...] * (z_ref[...] - m_ref[...])
             / jnp.sqrt(v_ref[...] + BN_EPS) + be_ref[...])
        valid = lax.broadcasted_iota(jnp.int32, (P, D), 0) < N
        o_ref[...] = jnp.where(valid, h, 0.0)
    return pl.pallas_call(
        body,
        out_shape=jax.ShapeDtypeStruct((P, D), jnp.float32),
    )(z, mean.reshape(1, D), var.reshape(1, D),
      gamma.reshape(1, D), beta.reshape(1, D))


def _tc_fc(pool_parts, fc_w, fc_b):
    def body(pp_ref, w_ref, b_ref, o_ref):
        pooled = pp_ref[0, :G, :] + pp_ref[1, :G, :]
        o_ref[...] = jnp.maximum(
            jnp.dot(pooled, w_ref[...], preferred_element_type=jnp.float32)
            + b_ref[...], 0.0)
    return pl.pallas_call(
        body,
        out_shape=jax.ShapeDtypeStruct((G, OUT), jnp.float32),
    )(pool_parts, fc_w, fc_b.reshape(1, OUT))


def kernel(x, edge_index, batch, params):
    layers = params["layers"]

    src, dst = edge_index[0], edge_index[1]
    perm = jnp.argsort(dst, stable=True)
    src_s = src[perm]
    dst_s = dst[perm]
    pad_e = EPAD - E
    src_pad = jnp.concatenate([src_s, jnp.full((pad_e,), ZROW, jnp.int32)])
    dst_pad = jnp.concatenate([dst_s, jnp.full((pad_e,), TRASH, jnp.int32)])
    edges = jnp.stack([src_pad, dst_pad]).reshape(2, NW, NCHUNK, CHUNK)

    x_pad = jnp.pad(x, ((0, P - N), (0, 0)))
    pool_dst = jnp.concatenate(
        [batch, jnp.full((P - N,), GTRASH, jnp.int32)]).reshape(
        NW, NCH_P, CHUNK_P)

    h = x_pad
    for l in range(5):
        w1, b1, w2, b2 = layers[l]["mlp"]
        if l == 0:
            parts4 = [_sc_fold(h[:, k * D:(k + 1) * D], edges)
                      for k in range(DF // D)]
            z = _tc_mlp1(h, parts4, w1, b1, w2, b2)
        else:
            parts = _sc_fold(h, edges)
            z = _tc_mlp(h, parts, w1, b1, w2, b2)
        zn = z[:N]
        mean = jnp.mean(zn, axis=0)
        var = jnp.var(zn, axis=0)
        h = _tc_norm(z, mean, var, layers[l]["gamma"], layers[l]["beta"])

    pool_parts = _sc_pool(h, pool_dst)
    return _tc_fc(pool_parts, params["fc_w"], params["fc_b"])
